# baseline (speedup 1.0000x reference)
import jax
import jax.numpy as jnp
from jax.experimental import pallas as pl

B, N, DEG, NP = 64, 64, 6, 512
F_M = 37
F_P = 480
BB = 4


def _dot(a, b):
    return jax.lax.dot_general(a, b, (((1,), (0,)), ((), ())),
                               preferred_element_type=jnp.float32)


def _gdot(G, v):
    return jax.lax.dot_general(G, v, (((1,), (0,)), ((), ())),
                               precision=jax.lax.Precision.HIGHEST,
                               preferred_element_type=jnp.float32)


def _body(m_atoms_ref, m_bonds_ref, m_edges_ref, p_atoms_ref, p_edges_ref,
          W_m1_ref, b_m1_ref, W_m2_ref, b_m2_ref, W_go_ref, b_go_ref,
          W_p1_ref, b_p1_ref, W_p2_ref, b_p2_ref, W_gop_ref, b_gop_ref,
          W_fc1_ref, b_fc1_ref, W_fc3_ref, b_fc3_ref, W_fc2_ref, b_fc2_ref,
          out_ref):
    W_m1 = W_m1_ref[...]
    W_m2 = W_m2_ref[...]
    W_go = W_go_ref[...]
    col = jax.lax.broadcasted_iota(jnp.int32, (N, N), 1)

    a0s = [m_atoms_ref[i] for i in range(BB)]
    bsums = [jnp.sum(m_bonds_ref[i], axis=1) for i in range(BB)]
    Gs = []
    for i in range(BB):
        edges = m_edges_ref[i]
        G = jnp.zeros((N, N), jnp.float32)
        for d in range(DEG):
            G = G + (edges[:, d][:, None] == col).astype(jnp.float32)
        Gs.append(G)

    adjs = [p_edges_ref[i] for i in range(BB)]
    t1s = [_dot(adjs[i], p_atoms_ref[i]) for i in range(BB)]
    p1s = [jax.nn.relu(_dot(t1s[i], W_p1_ref[...]) + b_p1_ref[...][None, :])
           for i in range(BB)]
    h1s = [jax.nn.relu(_dot(a0s[i] + _gdot(Gs[i], a0s[i]), W_m1[:F_M])
                       + _dot(bsums[i], W_m1[F_M:]) + b_m1_ref[...][None, :])
           for i in range(BB)]
    t2s = [_dot(adjs[i], p1s[i]) for i in range(BB)]
    h2s = [jax.nn.relu(_dot(h1s[i] + _gdot(Gs[i], h1s[i]), W_m2[:128])
                       + _dot(bsums[i], W_m2[128:]) + b_m2_ref[...][None, :])
           for i in range(BB)]
    p2s = [jax.nn.relu(_dot(t2s[i], W_p2_ref[...]) + b_p2_ref[...][None, :])
           for i in range(BB)]
    gos = [jnp.tanh(_dot(h2s[i], W_go[:128]) + _dot(bsums[i], W_go[128:])
                    + b_go_ref[...][None, :])
           for i in range(BB)]
    gs = [jnp.tanh(_dot(p2s[i], W_gop_ref[...]) + b_gop_ref[...][None, :])
          for i in range(BB)]
    fp_m = jnp.concatenate([jnp.sum(gos[i], axis=0, keepdims=True)
                            for i in range(BB)], axis=0)
    fp_p = jnp.concatenate([jnp.sum(gs[i], axis=0, keepdims=True)
                            for i in range(BB)], axis=0)
    W_fc1 = W_fc1_ref[...]
    tmp = _dot(fp_m, W_fc1[:128]) + _dot(fp_p, W_fc1[128:]) \
        + b_fc1_ref[...][None, :]
    tmp1 = _dot(tmp, W_fc3_ref[...]) + b_fc3_ref[...][None, :]
    o = jax.nn.sigmoid(_dot(tmp1, W_fc2_ref[...]) + b_fc2_ref[...][None, :])
    out_ref[...] = o.reshape(BB, 1, 1)


@jax.jit
def kernel(m_atoms, m_bonds, m_edges, p_atoms, p_edges,
           W_m1, b_m1, W_m2, b_m2, W_go, b_go,
           W_p1, b_p1, W_p2, b_p2, W_gop, b_gop,
           W_fc1, b_fc1, W_fc3, b_fc3, W_fc2, b_fc2):
    whole = lambda *s: pl.BlockSpec(s, lambda b: (0,) * len(s))
    per_b3 = lambda d1, d2: pl.BlockSpec((BB, d1, d2), lambda b: (b, 0, 0))

    out = pl.pallas_call(
        _body,
        grid=(B // BB,),
        in_specs=[
            per_b3(N, F_M),
            pl.BlockSpec((BB, N, DEG, 6), lambda b: (b, 0, 0, 0)),
            per_b3(N, DEG),
            per_b3(NP, F_P),
            per_b3(NP, NP),
            whole(43, 128), whole(128),
            whole(134, 128), whole(128),
            whole(134, 128), whole(128),
            whole(F_P, 200), whole(200),
            whole(200, 100), whole(100),
            whole(100, 128), whole(128),
            whole(256, 100), whole(100),
            whole(100, 100), whole(100),
            whole(100, 1), whole(1),
        ],
        out_specs=pl.BlockSpec((BB, 1, 1), lambda b: (b, 0, 0)),
        out_shape=jax.ShapeDtypeStruct((B, 1, 1), jnp.float32),
    )(m_atoms, m_bonds, m_edges, p_atoms, p_edges,
      W_m1, b_m1, W_m2, b_m2, W_go, b_go,
      W_p1, b_p1, W_p2, b_p2, W_gop, b_gop,
      W_fc1, b_fc1, W_fc3, b_fc3, W_fc2, b_fc2)
    return out.reshape(B, 1)


# speedup vs baseline: 1.0686x; 1.0686x over previous
import jax
import jax.numpy as jnp
from jax.experimental import pallas as pl

B, N, DEG, NP = 64, 64, 6, 512
F_M = 37
F_P = 480
BB = 4


def _dot(a, b):
    return jax.lax.dot_general(a, b, (((1,), (0,)), ((), ())),
                               preferred_element_type=jnp.float32)


def _gdot(G, v):
    return jax.lax.dot_general(G, v, (((1,), (0,)), ((), ())),
                               precision=jax.lax.Precision.HIGHEST,
                               preferred_element_type=jnp.float32)


def _body(m_atoms_ref, m_bonds_ref, m_edges_ref, p_atoms_ref, p_edges_ref,
          W_m1_ref, b_m1_ref, W_m2_ref, b_m2_ref, W_go_ref, b_go_ref,
          W_p1_ref, b_p1_ref, W_p2_ref, b_p2_ref, W_gop_ref, b_gop_ref,
          fp_m_ref, fp_p_ref):
    W_m1 = W_m1_ref[...]
    W_m2 = W_m2_ref[...]
    W_go = W_go_ref[...]
    col = jax.lax.broadcasted_iota(jnp.int32, (N, N), 1)

    a0s = [m_atoms_ref[i] for i in range(BB)]
    bsums = [jnp.sum(m_bonds_ref[i], axis=1) for i in range(BB)]
    Gs = []
    for i in range(BB):
        edges = m_edges_ref[i]
        G = jnp.zeros((N, N), jnp.float32)
        for d in range(DEG):
            G = G + (edges[:, d][:, None] == col).astype(jnp.float32)
        Gs.append(G)

    adjs = [p_edges_ref[i] for i in range(BB)]
    t1s = [_dot(adjs[i], p_atoms_ref[i]) for i in range(BB)]
    p1s = [jax.nn.relu(_dot(t1s[i], W_p1_ref[...]) + b_p1_ref[...][None, :])
           for i in range(BB)]
    h1s = [jax.nn.relu(_dot(a0s[i] + _gdot(Gs[i], a0s[i]), W_m1[:F_M])
                       + _dot(bsums[i], W_m1[F_M:]) + b_m1_ref[...][None, :])
           for i in range(BB)]
    t2s = [_dot(adjs[i], p1s[i]) for i in range(BB)]
    h2s = [jax.nn.relu(_dot(h1s[i] + _gdot(Gs[i], h1s[i]), W_m2[:128])
                       + _dot(bsums[i], W_m2[128:]) + b_m2_ref[...][None, :])
           for i in range(BB)]
    p2s = [jax.nn.relu(_dot(t2s[i], W_p2_ref[...]) + b_p2_ref[...][None, :])
           for i in range(BB)]
    gos = [jnp.tanh(_dot(h2s[i], W_go[:128]) + _dot(bsums[i], W_go[128:])
                    + b_go_ref[...][None, :])
           for i in range(BB)]
    gs = [jnp.tanh(_dot(p2s[i], W_gop_ref[...]) + b_gop_ref[...][None, :])
          for i in range(BB)]
    for i in range(BB):
        fp_p_ref[i] = jnp.sum(gs[i], axis=0, keepdims=True)
        fp_m_ref[i] = jnp.sum(gos[i], axis=0, keepdims=True)


def _head_body(fp_m_ref, fp_p_ref, W_fc1_ref, b_fc1_ref, W_fc3_ref,
               b_fc3_ref, W_fc2_ref, b_fc2_ref, out_ref):
    W_fc1 = W_fc1_ref[...]
    tmp = _dot(fp_m_ref[...], W_fc1[:128]) + _dot(fp_p_ref[...], W_fc1[128:]) \
        + b_fc1_ref[...][None, :]
    tmp1 = _dot(tmp, W_fc3_ref[...]) + b_fc3_ref[...][None, :]
    out_ref[...] = jax.nn.sigmoid(_dot(tmp1, W_fc2_ref[...])
                                  + b_fc2_ref[...][None, :])


@jax.jit
def kernel(m_atoms, m_bonds, m_edges, p_atoms, p_edges,
           W_m1, b_m1, W_m2, b_m2, W_go, b_go,
           W_p1, b_p1, W_p2, b_p2, W_gop, b_gop,
           W_fc1, b_fc1, W_fc3, b_fc3, W_fc2, b_fc2):
    whole = lambda *s: pl.BlockSpec(s, lambda b: (0,) * len(s))
    per_b3 = lambda d1, d2: pl.BlockSpec((BB, d1, d2), lambda b: (b, 0, 0))

    fp_m, fp_p = pl.pallas_call(
        _body,
        grid=(B // BB,),
        in_specs=[
            per_b3(N, F_M),
            pl.BlockSpec((BB, N, DEG, 6), lambda b: (b, 0, 0, 0)),
            per_b3(N, DEG),
            per_b3(NP, F_P),
            per_b3(NP, NP),
            whole(43, 128), whole(128),
            whole(134, 128), whole(128),
            whole(134, 128), whole(128),
            whole(F_P, 200), whole(200),
            whole(200, 100), whole(100),
            whole(100, 128), whole(128),
        ],
        out_specs=[pl.BlockSpec((BB, 1, 128), lambda b: (b, 0, 0)),
                   pl.BlockSpec((BB, 1, 128), lambda b: (b, 0, 0))],
        out_shape=[jax.ShapeDtypeStruct((B, 1, 128), jnp.float32),
                   jax.ShapeDtypeStruct((B, 1, 128), jnp.float32)],
    )(m_atoms, m_bonds, m_edges, p_atoms, p_edges,
      W_m1, b_m1, W_m2, b_m2, W_go, b_go,
      W_p1, b_p1, W_p2, b_p2, W_gop, b_gop)

    out = pl.pallas_call(
        _head_body,
        out_shape=jax.ShapeDtypeStruct((B, 1), jnp.float32),
    )(fp_m.reshape(B, 128), fp_p.reshape(B, 128),
      W_fc1, b_fc1, W_fc3, b_fc3, W_fc2, b_fc2)
    return out
